# baseline (speedup 1.0000x reference)
import jax
import jax.numpy as jnp
import numpy as np
from jax.experimental import pallas as pl
from jax.experimental.pallas import tpu as pltpu

_CHUNK = 256


def _conv_body(x_ref, p_ref, w_ref, b_ref, o_ref):
    l = x_ref.shape[2]
    half = _CHUNK // 2
    p = p_ref[...]
    for i in range(x_ref.shape[0]):
        xt = x_ref[i]
        evens, odds = [], []
        for k in range(l // _CHUNK):
            s = jnp.dot(xt[:, k * _CHUNK:(k + 1) * _CHUNK], p,
                        preferred_element_type=jnp.float32).astype(jnp.bfloat16)
            evens.append(s[:, 0:half])
            odds.append(s[:, half:_CHUNK])
        even = jnp.concatenate(evens, axis=1)
        odd = jnp.concatenate(odds, axis=1)
        odd_prev = jnp.concatenate([odd[:, :1], odd[:, :-1]], axis=1)
        acc = jnp.dot(w_ref[0], odd_prev, preferred_element_type=jnp.float32)
        acc = acc + jnp.dot(w_ref[1], even,
                            preferred_element_type=jnp.float32)
        acc = acc + jnp.dot(w_ref[2], odd, preferred_element_type=jnp.float32)
        o_ref[i] = (acc + b_ref[...]).astype(o_ref.dtype)


def kernel(x, conv_w, conv_b):
    B, Cin, L = x.shape
    Cout = conv_w.shape[0]
    assert conv_w.shape == (Cout, Cin, 3)
    assert L % _CHUNK == 0
    Lout = L // 2

    half = _CHUNK // 2
    m = jax.lax.broadcasted_iota(jnp.int32, (_CHUNK, _CHUNK), 0)
    col = jax.lax.broadcasted_iota(jnp.int32, (_CHUNK, _CHUNK), 1)
    sel = m == jnp.where(col < half, 2 * col, 2 * (col - half) + 1)
    p2 = sel.astype(jnp.float32)

    w_k = jnp.transpose(conv_w, (2, 0, 1)).astype(jnp.bfloat16)
    b2 = conv_b.reshape(Cout, 1).astype(jnp.float32)

    bb = 4 if B % 4 == 0 else (2 if B % 2 == 0 else 1)
    return pl.pallas_call(
        _conv_body,
        out_shape=jax.ShapeDtypeStruct((B, Cout, Lout), x.dtype),
        grid=(B // bb,),
        in_specs=[
            pl.BlockSpec((bb, Cin, L), lambda b: (b, 0, 0)),
            pl.BlockSpec((_CHUNK, _CHUNK), lambda b: (0, 0)),
            pl.BlockSpec((3, Cout, Cin), lambda b: (0, 0, 0)),
            pl.BlockSpec((Cout, 1), lambda b: (0, 0)),
        ],
        out_specs=pl.BlockSpec((bb, Cout, Lout), lambda b: (b, 0, 0)),
        compiler_params=pltpu.CompilerParams(
            dimension_semantics=("parallel",),
            vmem_limit_bytes=64 * 1024 * 1024),
    )(x, p2, w_k, b2)


# speedup vs baseline: 1.0347x; 1.0347x over previous
import jax
import jax.numpy as jnp
import numpy as np
from jax.experimental import pallas as pl
from jax.experimental.pallas import tpu as pltpu

_CHUNK = 256


def _conv_body(x_ref, p_ref, w_ref, b_ref, o_ref):
    l = x_ref.shape[2]
    half = _CHUNK // 2
    p = p_ref[...]
    for i in range(x_ref.shape[0]):
        xt = x_ref[i]
        evens, odds = [], []
        for k in range(l // _CHUNK):
            s = jnp.dot(xt[:, k * _CHUNK:(k + 1) * _CHUNK], p,
                        preferred_element_type=jnp.float32).astype(jnp.bfloat16)
            evens.append(s[:, 0:half])
            odds.append(s[:, half:_CHUNK])
        even = jnp.concatenate(evens, axis=1)
        odd = jnp.concatenate(odds, axis=1)
        odd_prev = jnp.concatenate([odd[:, :1], odd[:, :-1]], axis=1)
        acc = jnp.dot(w_ref[0], odd_prev, preferred_element_type=jnp.float32)
        acc = acc + jnp.dot(w_ref[1], even,
                            preferred_element_type=jnp.float32)
        acc = acc + jnp.dot(w_ref[2], odd, preferred_element_type=jnp.float32)
        o_ref[i] = (acc + b_ref[...]).astype(o_ref.dtype)


def kernel(x, conv_w, conv_b):
    B, Cin, L = x.shape
    Cout = conv_w.shape[0]
    assert conv_w.shape == (Cout, Cin, 3)
    assert L % _CHUNK == 0
    Lout = L // 2

    half = _CHUNK // 2
    m = jax.lax.broadcasted_iota(jnp.int32, (_CHUNK, _CHUNK), 0)
    col = jax.lax.broadcasted_iota(jnp.int32, (_CHUNK, _CHUNK), 1)
    sel = m == jnp.where(col < half, 2 * col, 2 * (col - half) + 1)
    p2 = sel.astype(jnp.float32)

    w_k = jnp.transpose(conv_w, (2, 0, 1)).astype(jnp.bfloat16)
    b2 = conv_b.reshape(Cout, 1).astype(jnp.float32)

    bb = 2 if B % 2 == 0 else 1
    return pl.pallas_call(
        _conv_body,
        out_shape=jax.ShapeDtypeStruct((B, Cout, Lout), x.dtype),
        grid=(B // bb,),
        in_specs=[
            pl.BlockSpec((bb, Cin, L), lambda b: (b, 0, 0)),
            pl.BlockSpec((_CHUNK, _CHUNK), lambda b: (0, 0)),
            pl.BlockSpec((3, Cout, Cin), lambda b: (0, 0, 0)),
            pl.BlockSpec((Cout, 1), lambda b: (0, 0)),
        ],
        out_specs=pl.BlockSpec((bb, Cout, Lout), lambda b: (b, 0, 0)),
        compiler_params=pltpu.CompilerParams(
            dimension_semantics=("parallel",)),
    )(x, p2, w_k, b2)
